# baseline (speedup 1.0000x reference)
import functools

import jax
import jax.numpy as jnp
from jax import lax
from jax.experimental import pallas as pl
from jax.experimental.pallas import tpu as pltpu
from jax.experimental.pallas import tpu_sc as plsc

E = 8
NB = 2
B = 16
T = 32
H = 768
F = 3072
R = B * NB
GS = 4
G = 14
S = G * GS
S_PAD = 64


def _gate_kernel(x_ref, gw_ref, prob_ref, topv_ref, topi_ref, imp_ref):
    x = x_ref[...]
    xm = jnp.mean(x, axis=1)
    logits = jax.lax.dot_general(
        xm, gw_ref[...], (((1,), (1,)), ((), ())),
        preferred_element_type=jnp.float32)
    p = jax.nn.softmax(logits, axis=-1)
    prob_ref[...] = p
    cols = jax.lax.broadcasted_iota(jnp.int32, p.shape, 1)
    i1 = jnp.argmax(p, axis=-1)
    v1 = jnp.max(p, axis=-1)
    pm = jnp.where(cols == i1[:, None], -jnp.inf, p)
    i2 = jnp.argmax(pm, axis=-1)
    v2 = jnp.max(pm, axis=-1)
    topv_ref[...] = jnp.concatenate([v1[:, None], v2[:, None]], axis=1)
    topi_ref[...] = jnp.concatenate([i1[:, None], i2[:, None]], axis=1)
    imp = jnp.sum(p, axis=0, keepdims=True)
    m = jnp.mean(imp)
    var = jnp.sum((imp - m) ** 2) / (E - 1)
    imp_ref[...] = (var / (m * m)).reshape(1, 1)


L = 16


GE_OFF = 0
SB_OFF = L
SR_OFF = L + S_PAD
PACKED = L + 2 * S_PAD


def _route_sc_body(sel_hbm, tv_hbm, outi_hbm, outf_hbm,
                   selv, tvv, permv, outiv, spv, sem1, sem2):
    cid = lax.axis_index("c")
    sid = lax.axis_index("s")

    @pl.when((cid == 0) & (sid == 0))
    def _():
        cp1 = pltpu.make_async_copy(sel_hbm, selv, sem1)
        cp2 = pltpu.make_async_copy(tv_hbm, tvv, sem2)
        cp1.start()
        cp2.start()
        cp1.wait()
        cp2.wait()
        s0 = selv[pl.ds(0, L)]
        s1 = selv[pl.ds(L, L)]
        iota = lax.iota(jnp.int32, L)
        zero = iota * 0
        cnt, rank0, rank1 = [], zero, zero
        for e in range(E):
            m0 = (s0 == e).astype(jnp.int32)
            m1 = (s1 == e).astype(jnp.int32)
            c0 = jnp.sum(m0)
            cnt.append(c0 + jnp.sum(m1))
            rank0 = rank0 + m0 * (plsc.cumsum(m0) - m0)
            rank1 = rank1 + m1 * (plsc.cumsum(m1) - m1 + c0)
        starts, ngr, gstarts = [], [], []
        acc_s = 0
        acc_g = 0
        for e in range(E):
            starts.append(acc_s)
            gstarts.append(acc_g)
            ngr.append((cnt[e] + GS - 1) // GS)
            acc_s = acc_s + cnt[e]
            acc_g = acc_g + ngr[e]
        used = acc_g
        last_e = jnp.int32(0)
        for e in range(E):
            last_e = jnp.where(cnt[e] > 0, jnp.int32(e), last_e)
        st0, st1 = zero, zero
        for e in range(E):
            st0 = st0 + (s0 == e).astype(jnp.int32) * starts[e]
            st1 = st1 + (s1 == e).astype(jnp.int32) * starts[e]
        plsc.store_scatter(permv, [st0 + rank0], iota)
        plsc.store_scatter(permv, [st1 + rank1], iota + L)
        def expert_of(g):
            ev = g * 0
            for e in range(E):
                m = (g >= gstarts[e]) & (g < gstarts[e] + ngr[e])
                ev = ev + m.astype(jnp.int32) * e
            return ev + (g >= used).astype(jnp.int32) * last_e

        outiv[pl.ds(GE_OFF, L)] = expert_of(iota)
        for c in range(S_PAD // L):
            sids = iota + L * c
            gg = sids // GS
            e_s = expert_of(gg)
            gst, cts, sts = zero, zero, zero
            for e in range(E):
                m = (e_s == e).astype(jnp.int32)
                gst = gst + m * gstarts[e]
                cts = cts + m * cnt[e]
                sts = sts + m * starts[e]
            j = (gg - gst) * GS + sids % GS
            valid = (j < cts).astype(jnp.int32)
            sidx = jnp.clip(sts + j, 0, R - 1)
            rows = plsc.load_gather(permv, [sidx])
            probs = plsc.load_gather(tvv, [rows])
            outiv[pl.ds(SR_OFF + L * c, L)] = rows * valid + (1 - valid) * R
            outiv[pl.ds(SB_OFF + L * c, L)] = (rows // NB) * valid
            spv[pl.ds(L * c, L)] = probs * valid.astype(jnp.float32)
        cp3 = pltpu.make_async_copy(outiv, outi_hbm, sem1)
        cp4 = pltpu.make_async_copy(spv, outf_hbm, sem2)
        cp3.start()
        cp4.start()
        cp3.wait()
        cp4.wait()


def _route_sc(sel, beam_scores):
    return pl.kernel(
        _route_sc_body,
        out_type=(
            jax.ShapeDtypeStruct((PACKED,), jnp.int32),
            jax.ShapeDtypeStruct((S_PAD,), jnp.float32),
        ),
        mesh=plsc.VectorSubcoreMesh(core_axis_name="c", subcore_axis_name="s"),
        compiler_params=pltpu.CompilerParams(needs_layout_passes=False),
        scratch_types=[
            pltpu.VMEM((R,), jnp.int32),
            pltpu.VMEM((R,), jnp.float32),
            pltpu.VMEM((R,), jnp.int32),
            pltpu.VMEM((PACKED,), jnp.int32),
            pltpu.VMEM((S_PAD,), jnp.float32),
            pltpu.SemaphoreType.DMA,
            pltpu.SemaphoreType.DMA,
        ],
    )(sel, beam_scores)


def _ffn_kernel(ci_ref, sp_ref,
                x_ref, w1_ref, b1_ref, w2_ref, b2_ref, out_ref):
    g = pl.program_id(0)
    xg = jnp.concatenate(
        [x_ref[ci_ref[SB_OFF + GS * g + s]] for s in range(GS)],
        axis=0)
    h = jax.lax.dot_general(
        xg.astype(jnp.bfloat16), w1_ref[0].astype(jnp.bfloat16),
        (((1,), (1,)), ((), ())),
        preferred_element_type=jnp.float32)
    h = jax.nn.gelu(h + b1_ref[0, 0])
    part = jax.lax.dot_general(
        h.astype(jnp.bfloat16), w2_ref[0].astype(jnp.bfloat16),
        (((1,), (1,)), ((), ())),
        preferred_element_type=jnp.float32)
    part = part + b2_ref[0, 0]
    for s in range(GS):
        r = GS * g + s
        out_ref[ci_ref[SR_OFF + r]] = sp_ref[r] * part[T * s:T * (s + 1)]


@jax.jit
def kernel(x, gate_w, w1, b1, w2, b2):
    prob, topv, topi, imp = pl.pallas_call(
        _gate_kernel,
        out_shape=(
            jax.ShapeDtypeStruct((B, E), jnp.float32),
            jax.ShapeDtypeStruct((B, NB), jnp.float32),
            jax.ShapeDtypeStruct((B, NB), jnp.int32),
            jax.ShapeDtypeStruct((1, 1), jnp.float32),
        ),
    )(x, gate_w)

    sel = topi.reshape(R)
    beam_scores = topv.reshape(R)
    expert_route = sel[:, None]
    beam_idx = jnp.arange(R, dtype=jnp.int32)
    importance_loss = imp[0, 0]

    packed_i, slot_prob = _route_sc(sel, beam_scores)

    b1r = b1.reshape(E, 1, F)
    b2r = b2.reshape(E, 1, H)

    grid_spec = pltpu.PrefetchScalarGridSpec(
        num_scalar_prefetch=2,
        grid=(G,),
        in_specs=[
            pl.BlockSpec((B, T, H), lambda g, ci, sp: (0, 0, 0)),
            pl.BlockSpec((1, F, H), lambda g, ci, sp: (ci[g], 0, 0)),
            pl.BlockSpec((1, 1, F), lambda g, ci, sp: (ci[g], 0, 0)),
            pl.BlockSpec((1, H, F), lambda g, ci, sp: (ci[g], 0, 0)),
            pl.BlockSpec((1, 1, H), lambda g, ci, sp: (ci[g], 0, 0)),
        ],
        out_specs=pl.BlockSpec((R + 1, T, H), lambda g, ci, sp: (0, 0, 0)),
    )
    padded = pl.pallas_call(
        _ffn_kernel,
        grid_spec=grid_spec,
        out_shape=jax.ShapeDtypeStruct((R + 1, T, H), jnp.float32),
        compiler_params=pltpu.CompilerParams(
            vmem_limit_bytes=60000 * 1024),
    )(packed_i, slot_prob, x, w1, b1r, w2, b2r)
    out = padded[:R]

    return out, beam_scores, expert_route, beam_idx, importance_loss


# speedup vs baseline: 1.1574x; 1.1574x over previous
import functools

import jax
import jax.numpy as jnp
from jax import lax
from jax.experimental import pallas as pl
from jax.experimental.pallas import tpu as pltpu
from jax.experimental.pallas import tpu_sc as plsc

E = 8
NB = 2
B = 16
T = 32
H = 768
F = 3072
R = B * NB
GS = 4
G = 14
S = G * GS
S_PAD = 64


def _gate_kernel(x_ref, gw_ref, prob_ref, topv_ref, topi_ref, imp_ref):
    x = x_ref[...]
    xm = jnp.mean(x, axis=1)
    logits = jax.lax.dot_general(
        xm, gw_ref[...], (((1,), (1,)), ((), ())),
        preferred_element_type=jnp.float32)
    p = jax.nn.softmax(logits, axis=-1)
    prob_ref[...] = p
    cols = jax.lax.broadcasted_iota(jnp.int32, p.shape, 1)
    i1 = jnp.argmax(p, axis=-1)
    v1 = jnp.max(p, axis=-1)
    pm = jnp.where(cols == i1[:, None], -jnp.inf, p)
    i2 = jnp.argmax(pm, axis=-1)
    v2 = jnp.max(pm, axis=-1)
    topv_ref[...] = jnp.concatenate([v1[:, None], v2[:, None]], axis=1)
    topi_ref[...] = jnp.concatenate([i1[:, None], i2[:, None]], axis=1)
    imp = jnp.sum(p, axis=0, keepdims=True)
    m = jnp.mean(imp)
    var = jnp.sum((imp - m) ** 2) / (E - 1)
    imp_ref[...] = (var / (m * m)).reshape(1, 1)


L = 16


GN_OFF = 0
SB_OFF = L
SR_OFF = L + S_PAD
PACKED = L + 2 * S_PAD
QMAX = (B + GS - 1) // GS


def _route_sc_body(sel_hbm, tv_hbm, outi_hbm, outf_hbm,
                   selv, tvv, permv, outiv, spv, sem1, sem2):
    cid = lax.axis_index("c")
    sid = lax.axis_index("s")

    @pl.when((cid == 0) & (sid == 0))
    def _():
        cp1 = pltpu.make_async_copy(sel_hbm, selv, sem1)
        cp2 = pltpu.make_async_copy(tv_hbm, tvv, sem2)
        cp1.start()
        cp2.start()
        cp1.wait()
        cp2.wait()
        s0 = selv[pl.ds(0, L)]
        s1 = selv[pl.ds(L, L)]
        iota = lax.iota(jnp.int32, L)
        zero = iota * 0
        cnt, rank0, rank1 = [], zero, zero
        for e in range(E):
            m0 = (s0 == e).astype(jnp.int32)
            m1 = (s1 == e).astype(jnp.int32)
            c0 = jnp.sum(m0)
            cnt.append(c0 + jnp.sum(m1))
            rank0 = rank0 + m0 * (plsc.cumsum(m0) - m0)
            rank1 = rank1 + m1 * (plsc.cumsum(m1) - m1 + c0)
        starts, ngr, gstarts = [], [], []
        acc_s = 0
        acc_g = 0
        for e in range(E):
            starts.append(acc_s)
            gstarts.append(acc_g)
            ngr.append((cnt[e] + GS - 1) // GS)
            acc_s = acc_s + cnt[e]
            acc_g = acc_g + ngr[e]
        used = acc_g
        last_e = jnp.int32(0)
        for e in range(E):
            last_e = jnp.where(cnt[e] > 0, jnp.int32(e), last_e)
        st0, st1 = zero, zero
        for e in range(E):
            st0 = st0 + (s0 == e).astype(jnp.int32) * starts[e]
            st1 = st1 + (s1 == e).astype(jnp.int32) * starts[e]
        plsc.store_scatter(permv, [st0 + rank0], iota)
        plsc.store_scatter(permv, [st1 + rank1], iota + L)
        def expert_of(g):
            ev = g * 0
            for e in range(E):
                m = (g >= gstarts[e]) & (g < gstarts[e] + ngr[e])
                ev = ev + m.astype(jnp.int32) * e
            return ev + (g >= used).astype(jnp.int32) * last_e

        gn = zero
        for e in range(E):
            gn = gn + (iota == e).astype(jnp.int32) * gstarts[e]
            gn = gn + (iota == E + e).astype(jnp.int32) * ngr[e]
        outiv[pl.ds(GN_OFF, L)] = gn
        for c in range(S_PAD // L):
            sids = iota + L * c
            gg = sids // GS
            e_s = expert_of(gg)
            gst, cts, sts = zero, zero, zero
            for e in range(E):
                m = (e_s == e).astype(jnp.int32)
                gst = gst + m * gstarts[e]
                cts = cts + m * cnt[e]
                sts = sts + m * starts[e]
            j = (gg - gst) * GS + sids % GS
            valid = (j < cts).astype(jnp.int32)
            sidx = jnp.clip(sts + j, 0, R - 1)
            rows = plsc.load_gather(permv, [sidx])
            probs = plsc.load_gather(tvv, [rows])
            outiv[pl.ds(SR_OFF + L * c, L)] = rows * valid + (1 - valid) * R
            outiv[pl.ds(SB_OFF + L * c, L)] = (rows // NB) * valid
            spv[pl.ds(L * c, L)] = probs * valid.astype(jnp.float32)
        cp3 = pltpu.make_async_copy(outiv, outi_hbm, sem1)
        cp4 = pltpu.make_async_copy(spv, outf_hbm, sem2)
        cp3.start()
        cp4.start()
        cp3.wait()
        cp4.wait()


def _route_sc(sel, beam_scores):
    return pl.kernel(
        _route_sc_body,
        out_type=(
            jax.ShapeDtypeStruct((PACKED,), jnp.int32),
            jax.ShapeDtypeStruct((S_PAD,), jnp.float32),
        ),
        mesh=plsc.VectorSubcoreMesh(core_axis_name="c", subcore_axis_name="s"),
        compiler_params=pltpu.CompilerParams(needs_layout_passes=False),
        scratch_types=[
            pltpu.VMEM((R,), jnp.int32),
            pltpu.VMEM((R,), jnp.float32),
            pltpu.VMEM((R,), jnp.int32),
            pltpu.VMEM((PACKED,), jnp.int32),
            pltpu.VMEM((S_PAD,), jnp.float32),
            pltpu.SemaphoreType.DMA,
            pltpu.SemaphoreType.DMA,
        ],
    )(sel, beam_scores)


def _ffn_kernel(ci_ref, sp_ref,
                x_ref, w1_ref, b1_ref, w2_ref, b2_ref, out_ref):
    e = pl.program_id(0)
    gst = ci_ref[GN_OFF + e]
    ng = ci_ref[GN_OFF + E + e]
    w1b = w1_ref[0].astype(jnp.bfloat16)
    w2b = w2_ref[0].astype(jnp.bfloat16)
    for q in range(QMAX):

        @pl.when(q < ng)
        def _():
            g = gst + q
            xg = jnp.concatenate(
                [x_ref[ci_ref[SB_OFF + GS * g + s]] for s in range(GS)],
                axis=0)
            h = jax.lax.dot_general(
                xg.astype(jnp.bfloat16), w1b, (((1,), (1,)), ((), ())),
                preferred_element_type=jnp.float32)
            h = jax.nn.gelu(h + b1_ref[0, 0])
            part = jax.lax.dot_general(
                h.astype(jnp.bfloat16), w2b, (((1,), (1,)), ((), ())),
                preferred_element_type=jnp.float32)
            part = part + b2_ref[0, 0]
            for s in range(GS):
                r = GS * g + s
                out_ref[ci_ref[SR_OFF + r]] = (
                    sp_ref[r] * part[T * s:T * (s + 1)])


@jax.jit
def kernel(x, gate_w, w1, b1, w2, b2):
    prob, topv, topi, imp = pl.pallas_call(
        _gate_kernel,
        out_shape=(
            jax.ShapeDtypeStruct((B, E), jnp.float32),
            jax.ShapeDtypeStruct((B, NB), jnp.float32),
            jax.ShapeDtypeStruct((B, NB), jnp.int32),
            jax.ShapeDtypeStruct((1, 1), jnp.float32),
        ),
    )(x, gate_w)

    sel = topi.reshape(R)
    beam_scores = topv.reshape(R)
    expert_route = sel[:, None]
    beam_idx = jnp.arange(R, dtype=jnp.int32)
    importance_loss = imp[0, 0]

    packed_i, slot_prob = _route_sc(sel, beam_scores)

    b1r = b1.reshape(E, 1, F)
    b2r = b2.reshape(E, 1, H)

    grid_spec = pltpu.PrefetchScalarGridSpec(
        num_scalar_prefetch=2,
        grid=(E,),
        in_specs=[
            pl.BlockSpec((B, T, H), lambda e, ci, sp: (0, 0, 0)),
            pl.BlockSpec((1, F, H), lambda e, ci, sp: (e, 0, 0)),
            pl.BlockSpec((1, 1, F), lambda e, ci, sp: (e, 0, 0)),
            pl.BlockSpec((1, H, F), lambda e, ci, sp: (e, 0, 0)),
            pl.BlockSpec((1, 1, H), lambda e, ci, sp: (e, 0, 0)),
        ],
        out_specs=pl.BlockSpec((R + 1, T, H), lambda e, ci, sp: (0, 0, 0)),
    )
    padded = pl.pallas_call(
        _ffn_kernel,
        grid_spec=grid_spec,
        out_shape=jax.ShapeDtypeStruct((R + 1, T, H), jnp.float32),
        compiler_params=pltpu.CompilerParams(
            vmem_limit_bytes=60000 * 1024),
    )(packed_i, slot_prob, x, w1, b1r, w2, b2r)
    out = padded[:R]

    return out, beam_scores, expert_route, beam_idx, importance_loss


# speedup vs baseline: 1.2375x; 1.0693x over previous
import functools

import jax
import jax.numpy as jnp
from jax import lax
from jax.experimental import pallas as pl
from jax.experimental.pallas import tpu as pltpu
from jax.experimental.pallas import tpu_sc as plsc

E = 8
NB = 2
B = 16
T = 32
H = 768
F = 3072
R = B * NB
GS = 4
G = 14
S = G * GS
S_PAD = 64


def _gate_kernel(x_ref, gw_ref, prob_ref, topv_ref, topi_ref, imp_ref):
    x = x_ref[...]
    xm = jnp.mean(x, axis=1)
    logits = jax.lax.dot_general(
        xm, gw_ref[...], (((1,), (1,)), ((), ())),
        preferred_element_type=jnp.float32)
    p = jax.nn.softmax(logits, axis=-1)
    prob_ref[...] = p
    cols = jax.lax.broadcasted_iota(jnp.int32, p.shape, 1)
    i1 = jnp.argmax(p, axis=-1)
    v1 = jnp.max(p, axis=-1)
    pm = jnp.where(cols == i1[:, None], -jnp.inf, p)
    i2 = jnp.argmax(pm, axis=-1)
    v2 = jnp.max(pm, axis=-1)
    topv_ref[...] = jnp.concatenate([v1[:, None], v2[:, None]], axis=1)
    topi_ref[...] = jnp.concatenate([i1[:, None], i2[:, None]], axis=1)
    imp = jnp.sum(p, axis=0, keepdims=True)
    m = jnp.mean(imp)
    var = jnp.sum((imp - m) ** 2) / (E - 1)
    imp_ref[...] = (var / (m * m)).reshape(1, 1)


L = 16


GN_OFF = 0
SB_OFF = L
SR_OFF = L + S_PAD
PACKED = L + 2 * S_PAD
QMAX = (B + GS - 1) // GS


def _route_sc_body(sel_hbm, tv_hbm, outi_hbm, outf_hbm,
                   selv, tvv, permv, outiv, spv, sem1, sem2):
    cid = lax.axis_index("c")
    sid = lax.axis_index("s")

    @pl.when((cid == 0) & (sid == 0))
    def _():
        cp1 = pltpu.make_async_copy(sel_hbm, selv, sem1)
        cp2 = pltpu.make_async_copy(tv_hbm, tvv, sem2)
        cp1.start()
        cp2.start()
        cp1.wait()
        cp2.wait()
        s0 = selv[pl.ds(0, L)]
        s1 = selv[pl.ds(L, L)]
        iota = lax.iota(jnp.int32, L)
        zero = iota * 0
        cnt, rank0, rank1 = [], zero, zero
        for e in range(E):
            m0 = (s0 == e).astype(jnp.int32)
            m1 = (s1 == e).astype(jnp.int32)
            c0 = jnp.sum(m0)
            cnt.append(c0 + jnp.sum(m1))
            rank0 = rank0 + m0 * (plsc.cumsum(m0) - m0)
            rank1 = rank1 + m1 * (plsc.cumsum(m1) - m1 + c0)
        starts, ngr, gstarts = [], [], []
        acc_s = 0
        acc_g = 0
        for e in range(E):
            starts.append(acc_s)
            gstarts.append(acc_g)
            ngr.append((cnt[e] + GS - 1) // GS)
            acc_s = acc_s + cnt[e]
            acc_g = acc_g + ngr[e]
        used = acc_g
        last_e = jnp.int32(0)
        for e in range(E):
            last_e = jnp.where(cnt[e] > 0, jnp.int32(e), last_e)
        st0, st1 = zero, zero
        for e in range(E):
            st0 = st0 + (s0 == e).astype(jnp.int32) * starts[e]
            st1 = st1 + (s1 == e).astype(jnp.int32) * starts[e]
        plsc.store_scatter(permv, [st0 + rank0], iota)
        plsc.store_scatter(permv, [st1 + rank1], iota + L)
        def expert_of(g):
            ev = g * 0
            for e in range(E):
                m = (g >= gstarts[e]) & (g < gstarts[e] + ngr[e])
                ev = ev + m.astype(jnp.int32) * e
            return ev + (g >= used).astype(jnp.int32) * last_e

        gn = zero
        for e in range(E):
            gn = gn + (iota == e).astype(jnp.int32) * gstarts[e]
            gn = gn + (iota == E + e).astype(jnp.int32) * ngr[e]
        outiv[pl.ds(GN_OFF, L)] = gn
        for c in range(S_PAD // L):
            sids = iota + L * c
            gg = sids // GS
            e_s = expert_of(gg)
            gst, cts, sts = zero, zero, zero
            for e in range(E):
                m = (e_s == e).astype(jnp.int32)
                gst = gst + m * gstarts[e]
                cts = cts + m * cnt[e]
                sts = sts + m * starts[e]
            j = (gg - gst) * GS + sids % GS
            valid = (j < cts).astype(jnp.int32)
            sidx = jnp.clip(sts + j, 0, R - 1)
            rows = plsc.load_gather(permv, [sidx])
            probs = plsc.load_gather(tvv, [rows])
            outiv[pl.ds(SR_OFF + L * c, L)] = rows * valid + (1 - valid) * R
            outiv[pl.ds(SB_OFF + L * c, L)] = (rows // NB) * valid
            spv[pl.ds(L * c, L)] = probs * valid.astype(jnp.float32)
        cp3 = pltpu.make_async_copy(outiv, outi_hbm, sem1)
        cp4 = pltpu.make_async_copy(spv, outf_hbm, sem2)
        cp3.start()
        cp4.start()
        cp3.wait()
        cp4.wait()


def _route_sc(sel, beam_scores):
    return pl.kernel(
        _route_sc_body,
        out_type=(
            jax.ShapeDtypeStruct((PACKED,), jnp.int32),
            jax.ShapeDtypeStruct((S_PAD,), jnp.float32),
        ),
        mesh=plsc.VectorSubcoreMesh(core_axis_name="c", subcore_axis_name="s"),
        compiler_params=pltpu.CompilerParams(needs_layout_passes=False),
        scratch_types=[
            pltpu.VMEM((R,), jnp.int32),
            pltpu.VMEM((R,), jnp.float32),
            pltpu.VMEM((R,), jnp.int32),
            pltpu.VMEM((PACKED,), jnp.int32),
            pltpu.VMEM((S_PAD,), jnp.float32),
            pltpu.SemaphoreType.DMA,
            pltpu.SemaphoreType.DMA,
        ],
    )(sel, beam_scores)


def _ffn_kernel(ci_ref, sp_ref,
                x_ref, w1_ref, b1_ref, w2_ref, b2_ref, out_ref):
    e = pl.program_id(0)
    gst = ci_ref[GN_OFF + e]
    ng = ci_ref[GN_OFF + E + e]
    w1b = w1_ref[0].astype(jnp.bfloat16)
    w2b = w2_ref[0].astype(jnp.bfloat16)
    for q in range(QMAX):

        @pl.when(q < ng)
        def _():
            g = gst + q
            xg = jnp.concatenate(
                [x_ref[ci_ref[SB_OFF + GS * g + s]] for s in range(GS)],
                axis=0)
            h = jax.lax.dot_general(
                xg.astype(jnp.bfloat16), w1b, (((1,), (1,)), ((), ())),
                preferred_element_type=jnp.float32)
            h = jax.nn.gelu(h + b1_ref[0, 0])
            part = jax.lax.dot_general(
                h.astype(jnp.bfloat16), w2b, (((1,), (1,)), ((), ())),
                preferred_element_type=jnp.float32)
            part = part + b2_ref[0, 0]
            for s in range(GS):
                r = GS * g + s
                row = ci_ref[SR_OFF + r]

                @pl.when(row < R)
                def _():
                    out_ref[row] = sp_ref[r] * part[T * s:T * (s + 1)]


@jax.jit
def kernel(x, gate_w, w1, b1, w2, b2):
    prob, topv, topi, imp = pl.pallas_call(
        _gate_kernel,
        out_shape=(
            jax.ShapeDtypeStruct((B, E), jnp.float32),
            jax.ShapeDtypeStruct((B, NB), jnp.float32),
            jax.ShapeDtypeStruct((B, NB), jnp.int32),
            jax.ShapeDtypeStruct((1, 1), jnp.float32),
        ),
    )(x, gate_w)

    sel = topi.reshape(R)
    beam_scores = topv.reshape(R)
    expert_route = sel[:, None]
    beam_idx = jnp.arange(R, dtype=jnp.int32)
    importance_loss = imp[0, 0]

    packed_i, slot_prob = _route_sc(sel, beam_scores)

    b1r = b1.reshape(E, 1, F)
    b2r = b2.reshape(E, 1, H)

    grid_spec = pltpu.PrefetchScalarGridSpec(
        num_scalar_prefetch=2,
        grid=(E,),
        in_specs=[
            pl.BlockSpec((B, T, H), lambda e, ci, sp: (0, 0, 0)),
            pl.BlockSpec((1, F, H), lambda e, ci, sp: (e, 0, 0)),
            pl.BlockSpec((1, 1, F), lambda e, ci, sp: (e, 0, 0)),
            pl.BlockSpec((1, H, F), lambda e, ci, sp: (e, 0, 0)),
            pl.BlockSpec((1, 1, H), lambda e, ci, sp: (e, 0, 0)),
        ],
        out_specs=pl.BlockSpec((R, T, H), lambda e, ci, sp: (0, 0, 0)),
    )
    out = pl.pallas_call(
        _ffn_kernel,
        grid_spec=grid_spec,
        out_shape=jax.ShapeDtypeStruct((R, T, H), jnp.float32),
        compiler_params=pltpu.CompilerParams(
            vmem_limit_bytes=60000 * 1024),
    )(packed_i, slot_prob, x, w1, b1r, w2, b2r)

    return out, beam_scores, expert_route, beam_idx, importance_loss


# speedup vs baseline: 1.2399x; 1.0019x over previous
import functools

import jax
import jax.numpy as jnp
from jax import lax
from jax.experimental import pallas as pl
from jax.experimental.pallas import tpu as pltpu
from jax.experimental.pallas import tpu_sc as plsc

E = 8
NB = 2
B = 16
T = 32
H = 768
F = 3072
R = B * NB
GS = 4
G = 14
S = G * GS
S_PAD = 64
KC = 2
FH = F // KC


def _gate_kernel(x_ref, gw_ref, prob_ref, topv_ref, topi_ref, imp_ref):
    x = x_ref[...]
    xm = jnp.mean(x, axis=1)
    logits = jax.lax.dot_general(
        xm, gw_ref[...], (((1,), (1,)), ((), ())),
        preferred_element_type=jnp.float32)
    p = jax.nn.softmax(logits, axis=-1)
    prob_ref[...] = p
    cols = jax.lax.broadcasted_iota(jnp.int32, p.shape, 1)
    i1 = jnp.argmax(p, axis=-1)
    v1 = jnp.max(p, axis=-1)
    pm = jnp.where(cols == i1[:, None], -jnp.inf, p)
    i2 = jnp.argmax(pm, axis=-1)
    v2 = jnp.max(pm, axis=-1)
    topv_ref[...] = jnp.concatenate([v1[:, None], v2[:, None]], axis=1)
    topi_ref[...] = jnp.concatenate([i1[:, None], i2[:, None]], axis=1)
    imp = jnp.sum(p, axis=0, keepdims=True)
    m = jnp.mean(imp)
    var = jnp.sum((imp - m) ** 2) / (E - 1)
    imp_ref[...] = (var / (m * m)).reshape(1, 1)


L = 16


GN_OFF = 0
SB_OFF = L
SR_OFF = L + S_PAD
PACKED = L + 2 * S_PAD
QMAX = (B + GS - 1) // GS


def _route_sc_body(sel_hbm, tv_hbm, outi_hbm, outf_hbm,
                   selv, tvv, permv, outiv, spv, sem1, sem2):
    cid = lax.axis_index("c")
    sid = lax.axis_index("s")

    @pl.when((cid == 0) & (sid == 0))
    def _():
        cp1 = pltpu.make_async_copy(sel_hbm, selv, sem1)
        cp2 = pltpu.make_async_copy(tv_hbm, tvv, sem2)
        cp1.start()
        cp2.start()
        cp1.wait()
        cp2.wait()
        s0 = selv[pl.ds(0, L)]
        s1 = selv[pl.ds(L, L)]
        iota = lax.iota(jnp.int32, L)
        zero = iota * 0
        cnt, rank0, rank1 = [], zero, zero
        for e in range(E):
            m0 = (s0 == e).astype(jnp.int32)
            m1 = (s1 == e).astype(jnp.int32)
            c0 = jnp.sum(m0)
            cnt.append(c0 + jnp.sum(m1))
            rank0 = rank0 + m0 * (plsc.cumsum(m0) - m0)
            rank1 = rank1 + m1 * (plsc.cumsum(m1) - m1 + c0)
        starts, ngr, gstarts = [], [], []
        acc_s = 0
        acc_g = 0
        for e in range(E):
            starts.append(acc_s)
            gstarts.append(acc_g)
            ngr.append((cnt[e] + GS - 1) // GS)
            acc_s = acc_s + cnt[e]
            acc_g = acc_g + ngr[e]
        used = acc_g
        last_e = jnp.int32(0)
        for e in range(E):
            last_e = jnp.where(cnt[e] > 0, jnp.int32(e), last_e)
        st0, st1 = zero, zero
        for e in range(E):
            st0 = st0 + (s0 == e).astype(jnp.int32) * starts[e]
            st1 = st1 + (s1 == e).astype(jnp.int32) * starts[e]
        plsc.store_scatter(permv, [st0 + rank0], iota)
        plsc.store_scatter(permv, [st1 + rank1], iota + L)
        def expert_of(g):
            ev = g * 0
            for e in range(E):
                m = (g >= gstarts[e]) & (g < gstarts[e] + ngr[e])
                ev = ev + m.astype(jnp.int32) * e
            return ev + (g >= used).astype(jnp.int32) * last_e

        gn = zero
        for e in range(E):
            gn = gn + (iota == e).astype(jnp.int32) * gstarts[e]
            gn = gn + (iota == E + e).astype(jnp.int32) * ngr[e]
        outiv[pl.ds(GN_OFF, L)] = gn
        for c in range(S_PAD // L):
            sids = iota + L * c
            gg = sids // GS
            e_s = expert_of(gg)
            gst, cts, sts = zero, zero, zero
            for e in range(E):
                m = (e_s == e).astype(jnp.int32)
                gst = gst + m * gstarts[e]
                cts = cts + m * cnt[e]
                sts = sts + m * starts[e]
            j = (gg - gst) * GS + sids % GS
            valid = (j < cts).astype(jnp.int32)
            sidx = jnp.clip(sts + j, 0, R - 1)
            rows = plsc.load_gather(permv, [sidx])
            probs = plsc.load_gather(tvv, [rows])
            outiv[pl.ds(SR_OFF + L * c, L)] = rows * valid + (1 - valid) * R
            outiv[pl.ds(SB_OFF + L * c, L)] = (rows // NB) * valid
            spv[pl.ds(L * c, L)] = probs * valid.astype(jnp.float32)
        cp3 = pltpu.make_async_copy(outiv, outi_hbm, sem1)
        cp4 = pltpu.make_async_copy(spv, outf_hbm, sem2)
        cp3.start()
        cp4.start()
        cp3.wait()
        cp4.wait()


def _route_sc(sel, beam_scores):
    return pl.kernel(
        _route_sc_body,
        out_type=(
            jax.ShapeDtypeStruct((PACKED,), jnp.int32),
            jax.ShapeDtypeStruct((S_PAD,), jnp.float32),
        ),
        mesh=plsc.VectorSubcoreMesh(core_axis_name="c", subcore_axis_name="s"),
        compiler_params=pltpu.CompilerParams(needs_layout_passes=False),
        scratch_types=[
            pltpu.VMEM((R,), jnp.int32),
            pltpu.VMEM((R,), jnp.float32),
            pltpu.VMEM((R,), jnp.int32),
            pltpu.VMEM((PACKED,), jnp.int32),
            pltpu.VMEM((S_PAD,), jnp.float32),
            pltpu.SemaphoreType.DMA,
            pltpu.SemaphoreType.DMA,
        ],
    )(sel, beam_scores)


def _ffn_kernel(ci_ref, sp_ref,
                x_ref, w1_ref, b1_ref, w2_ref, b2_ref, out_ref, xs_ref):
    k = pl.program_id(0)
    e = pl.program_id(1)
    gst = ci_ref[GN_OFF + e]
    ng = ci_ref[GN_OFF + E + e]
    w1b = w1_ref[0].astype(jnp.bfloat16)
    w2b = w2_ref[0].astype(jnp.bfloat16)
    for q in range(QMAX):

        @pl.when(q < ng)
        def _():
            g = gst + q

            @pl.when(k == 0)
            def _():
                xs_ref[g] = jnp.concatenate(
                    [x_ref[ci_ref[SB_OFF + GS * g + s]] for s in range(GS)],
                    axis=0).astype(jnp.bfloat16)

            h = jax.lax.dot_general(
                xs_ref[g], w1b, (((1,), (1,)), ((), ())),
                preferred_element_type=jnp.float32)
            h = jax.nn.gelu(h + b1_ref[0, 0])
            part = jax.lax.dot_general(
                h.astype(jnp.bfloat16), w2b, (((1,), (1,)), ((), ())),
                preferred_element_type=jnp.float32)
            for s in range(GS):
                r = GS * g + s
                row = ci_ref[SR_OFF + r]

                @pl.when((row < R) & (k == 0))
                def _():
                    out_ref[row] = sp_ref[r] * (
                        part[T * s:T * (s + 1)] + b2_ref[0, 0])

                @pl.when((row < R) & (k != 0))
                def _():
                    out_ref[row] = out_ref[row] + (
                        sp_ref[r] * part[T * s:T * (s + 1)])


@jax.jit
def kernel(x, gate_w, w1, b1, w2, b2):
    prob, topv, topi, imp = pl.pallas_call(
        _gate_kernel,
        out_shape=(
            jax.ShapeDtypeStruct((B, E), jnp.float32),
            jax.ShapeDtypeStruct((B, NB), jnp.float32),
            jax.ShapeDtypeStruct((B, NB), jnp.int32),
            jax.ShapeDtypeStruct((1, 1), jnp.float32),
        ),
    )(x, gate_w)

    sel = topi.reshape(R)
    beam_scores = topv.reshape(R)
    expert_route = sel[:, None]
    beam_idx = jnp.arange(R, dtype=jnp.int32)
    importance_loss = imp[0, 0]

    packed_i, slot_prob = _route_sc(sel, beam_scores)

    b1r = b1.reshape(E, 1, F)
    b2r = b2.reshape(E, 1, H)

    grid_spec = pltpu.PrefetchScalarGridSpec(
        num_scalar_prefetch=2,
        grid=(KC, E),
        in_specs=[
            pl.BlockSpec((B, T, H), lambda k, e, ci, sp: (0, 0, 0)),
            pl.BlockSpec((1, FH, H), lambda k, e, ci, sp: (e, k, 0)),
            pl.BlockSpec((1, 1, FH), lambda k, e, ci, sp: (e, 0, k)),
            pl.BlockSpec((1, H, FH), lambda k, e, ci, sp: (e, 0, k)),
            pl.BlockSpec((1, 1, H), lambda k, e, ci, sp: (e, 0, 0)),
        ],
        out_specs=pl.BlockSpec((R, T, H), lambda k, e, ci, sp: (0, 0, 0)),
        scratch_shapes=[pltpu.VMEM((G, GS * T, H), jnp.bfloat16)],
    )
    out = pl.pallas_call(
        _ffn_kernel,
        grid_spec=grid_spec,
        out_shape=jax.ShapeDtypeStruct((R, T, H), jnp.float32),
        compiler_params=pltpu.CompilerParams(
            vmem_limit_bytes=60000 * 1024),
    )(packed_i, slot_prob, x, w1, b1r, w2, b2r)

    return out, beam_scores, expert_route, beam_idx, importance_loss


# speedup vs baseline: 1.2411x; 1.0010x over previous
import jax
import jax.numpy as jnp
from jax import lax
from jax.experimental import pallas as pl
from jax.experimental.pallas import tpu as pltpu
from jax.experimental.pallas import tpu_sc as plsc

E = 8
NB = 2
B = 16
T = 32
H = 768
F = 3072
R = B * NB
GS = 4
G = 14
S_PAD = 64
KC = 2
FH = F // KC


def _gate_kernel(x_ref, gw_ref, prob_ref, topv_ref, topi_ref, imp_ref):
    x = x_ref[...]
    xm = jnp.mean(x, axis=1)
    logits = jax.lax.dot_general(
        xm, gw_ref[...], (((1,), (1,)), ((), ())),
        preferred_element_type=jnp.float32)
    p = jax.nn.softmax(logits, axis=-1)
    prob_ref[...] = p
    cols = jax.lax.broadcasted_iota(jnp.int32, p.shape, 1)
    i1 = jnp.argmax(p, axis=-1)
    v1 = jnp.max(p, axis=-1)
    pm = jnp.where(cols == i1[:, None], -jnp.inf, p)
    i2 = jnp.argmax(pm, axis=-1)
    v2 = jnp.max(pm, axis=-1)
    topv_ref[...] = jnp.concatenate([v1[:, None], v2[:, None]], axis=1)
    topi_ref[...] = jnp.concatenate([i1[:, None], i2[:, None]], axis=1)
    imp = jnp.sum(p, axis=0, keepdims=True)
    m = jnp.mean(imp)
    var = jnp.sum((imp - m) ** 2) / (E - 1)
    imp_ref[...] = (var / (m * m)).reshape(1, 1)


L = 16


GN_OFF = 0
SB_OFF = L
SR_OFF = L + S_PAD
PACKED = L + 2 * S_PAD
QMAX = (B + GS - 1) // GS


def _route_sc_body(sel_hbm, tv_hbm, outi_hbm, outf_hbm,
                   selv, tvv, permv, outiv, spv, sem1, sem2):
    cid = lax.axis_index("c")
    sid = lax.axis_index("s")

    @pl.when((cid == 0) & (sid == 0))
    def _():
        cp1 = pltpu.make_async_copy(sel_hbm, selv, sem1)
        cp2 = pltpu.make_async_copy(tv_hbm, tvv, sem2)
        cp1.start()
        cp2.start()
        cp1.wait()
        cp2.wait()
        s0 = selv[pl.ds(0, L)]
        s1 = selv[pl.ds(L, L)]
        iota = lax.iota(jnp.int32, L)
        zero = iota * 0
        cnt, rank0, rank1 = [], zero, zero
        for e in range(E):
            m0 = (s0 == e).astype(jnp.int32)
            m1 = (s1 == e).astype(jnp.int32)
            c0 = jnp.sum(m0)
            cnt.append(c0 + jnp.sum(m1))
            rank0 = rank0 + m0 * (plsc.cumsum(m0) - m0)
            rank1 = rank1 + m1 * (plsc.cumsum(m1) - m1 + c0)
        starts, ngr, gstarts = [], [], []
        acc_s = 0
        acc_g = 0
        for e in range(E):
            starts.append(acc_s)
            gstarts.append(acc_g)
            ngr.append((cnt[e] + GS - 1) // GS)
            acc_s = acc_s + cnt[e]
            acc_g = acc_g + ngr[e]
        used = acc_g
        last_e = jnp.int32(0)
        for e in range(E):
            last_e = jnp.where(cnt[e] > 0, jnp.int32(e), last_e)
        st0, st1 = zero, zero
        for e in range(E):
            st0 = st0 + (s0 == e).astype(jnp.int32) * starts[e]
            st1 = st1 + (s1 == e).astype(jnp.int32) * starts[e]
        plsc.store_scatter(permv, [st0 + rank0], iota)
        plsc.store_scatter(permv, [st1 + rank1], iota + L)
        def expert_of(g):
            ev = g * 0
            for e in range(E):
                m = (g >= gstarts[e]) & (g < gstarts[e] + ngr[e])
                ev = ev + m.astype(jnp.int32) * e
            return ev + (g >= used).astype(jnp.int32) * last_e

        gn = zero
        for e in range(E):
            gn = gn + (iota == e).astype(jnp.int32) * gstarts[e]
            gn = gn + (iota == E + e).astype(jnp.int32) * ngr[e]
        outiv[pl.ds(GN_OFF, L)] = gn
        for c in range(S_PAD // L):
            sids = iota + L * c
            gg = sids // GS
            e_s = expert_of(gg)
            gst, cts, sts = zero, zero, zero
            for e in range(E):
                m = (e_s == e).astype(jnp.int32)
                gst = gst + m * gstarts[e]
                cts = cts + m * cnt[e]
                sts = sts + m * starts[e]
            j = (gg - gst) * GS + sids % GS
            valid = (j < cts).astype(jnp.int32)
            sidx = jnp.clip(sts + j, 0, R - 1)
            rows = plsc.load_gather(permv, [sidx])
            probs = plsc.load_gather(tvv, [rows])
            outiv[pl.ds(SR_OFF + L * c, L)] = rows * valid + (1 - valid) * R
            outiv[pl.ds(SB_OFF + L * c, L)] = (rows // NB) * valid
            spv[pl.ds(L * c, L)] = probs * valid.astype(jnp.float32)
        cp3 = pltpu.make_async_copy(outiv, outi_hbm, sem1)
        cp4 = pltpu.make_async_copy(spv, outf_hbm, sem2)
        cp3.start()
        cp4.start()
        cp3.wait()
        cp4.wait()


def _route_sc(sel, beam_scores):
    return pl.kernel(
        _route_sc_body,
        out_type=(
            jax.ShapeDtypeStruct((PACKED,), jnp.int32),
            jax.ShapeDtypeStruct((S_PAD,), jnp.float32),
        ),
        mesh=plsc.VectorSubcoreMesh(core_axis_name="c", subcore_axis_name="s"),
        compiler_params=pltpu.CompilerParams(needs_layout_passes=False),
        scratch_types=[
            pltpu.VMEM((R,), jnp.int32),
            pltpu.VMEM((R,), jnp.float32),
            pltpu.VMEM((R,), jnp.int32),
            pltpu.VMEM((PACKED,), jnp.int32),
            pltpu.VMEM((S_PAD,), jnp.float32),
            pltpu.SemaphoreType.DMA,
            pltpu.SemaphoreType.DMA,
        ],
    )(sel, beam_scores)


def _ffn_kernel(ci_ref, sp_ref,
                x_ref, w1_ref, b1_ref, w2_ref, b2_ref, out_ref, xs_ref):
    k = pl.program_id(0)
    e = pl.program_id(1)
    gst = ci_ref[GN_OFF + e]
    ng = ci_ref[GN_OFF + E + e]
    w1b = w1_ref[0].astype(jnp.bfloat16)
    w2b = w2_ref[0].astype(jnp.bfloat16)
    for q in range(QMAX):

        @pl.when(q < ng)
        def _():
            g = gst + q

            @pl.when(k == 0)
            def _():
                xs_ref[g] = jnp.concatenate(
                    [x_ref[ci_ref[SB_OFF + GS * g + s]] for s in range(GS)],
                    axis=0).astype(jnp.bfloat16)

            h = jax.lax.dot_general(
                xs_ref[g], w1b, (((1,), (1,)), ((), ())),
                preferred_element_type=jnp.float32)
            h = jax.nn.gelu(h + b1_ref[0, 0])
            part = jax.lax.dot_general(
                h.astype(jnp.bfloat16), w2b, (((1,), (1,)), ((), ())),
                preferred_element_type=jnp.float32)
            for s in range(GS):
                r = GS * g + s
                row = ci_ref[SR_OFF + r]

                @pl.when((row < R) & (k == 0))
                def _():
                    out_ref[row] = sp_ref[r] * (
                        part[T * s:T * (s + 1)] + b2_ref[0, 0])

                @pl.when((row < R) & (k != 0))
                def _():
                    out_ref[row] = out_ref[row] + (
                        sp_ref[r] * part[T * s:T * (s + 1)])


@jax.jit
def kernel(x, gate_w, w1, b1, w2, b2):
    prob, topv, topi, imp = pl.pallas_call(
        _gate_kernel,
        out_shape=(
            jax.ShapeDtypeStruct((B, E), jnp.float32),
            jax.ShapeDtypeStruct((B, NB), jnp.float32),
            jax.ShapeDtypeStruct((B, NB), jnp.int32),
            jax.ShapeDtypeStruct((1, 1), jnp.float32),
        ),
    )(x, gate_w)

    sel = topi.reshape(R)
    beam_scores = topv.reshape(R)
    expert_route = sel[:, None]
    beam_idx = jnp.arange(R, dtype=jnp.int32)
    importance_loss = imp[0, 0]

    packed_i, slot_prob = _route_sc(sel, beam_scores)

    b1r = b1.reshape(E, 1, F)
    b2r = b2.reshape(E, 1, H)

    grid_spec = pltpu.PrefetchScalarGridSpec(
        num_scalar_prefetch=2,
        grid=(KC, E),
        in_specs=[
            pl.BlockSpec((B, T, H), lambda k, e, ci, sp: (0, 0, 0)),
            pl.BlockSpec((1, FH, H), lambda k, e, ci, sp: (e, k, 0)),
            pl.BlockSpec((1, 1, FH), lambda k, e, ci, sp: (e, 0, k)),
            pl.BlockSpec((1, H, FH), lambda k, e, ci, sp: (e, 0, k)),
            pl.BlockSpec((1, 1, H), lambda k, e, ci, sp: (e, 0, 0)),
        ],
        out_specs=pl.BlockSpec((R, T, H), lambda k, e, ci, sp: (0, 0, 0)),
        scratch_shapes=[pltpu.VMEM((G, GS * T, H), jnp.bfloat16)],
    )
    out = pl.pallas_call(
        _ffn_kernel,
        grid_spec=grid_spec,
        out_shape=jax.ShapeDtypeStruct((R, T, H), jnp.float32),
        compiler_params=pltpu.CompilerParams(
            vmem_limit_bytes=60000 * 1024),
    )(packed_i, slot_prob, x, w1, b1r, w2, b2r)

    return out, beam_scores, expert_route, beam_idx, importance_loss


# speedup vs baseline: 1.2600x; 1.0152x over previous
import jax
import jax.numpy as jnp
from jax import lax
from jax.experimental import pallas as pl
from jax.experimental.pallas import tpu as pltpu
from jax.experimental.pallas import tpu_sc as plsc

E = 8
NB = 2
B = 16
T = 32
H = 768
F = 3072
R = B * NB
GS = 4
G = 14
S_PAD = 64
KC = 2
FH = F // KC


def _gate_kernel(x_ref, gw_ref, prob_ref, topv_ref, topi_ref, imp_ref):
    x = x_ref[...]
    xm = jnp.mean(x, axis=1)
    logits = jax.lax.dot_general(
        xm, gw_ref[...], (((1,), (1,)), ((), ())),
        preferred_element_type=jnp.float32)
    p = jax.nn.softmax(logits, axis=-1)
    prob_ref[...] = p
    cols = jax.lax.broadcasted_iota(jnp.int32, p.shape, 1)
    i1 = jnp.argmax(p, axis=-1)
    v1 = jnp.max(p, axis=-1)
    pm = jnp.where(cols == i1[:, None], -jnp.inf, p)
    i2 = jnp.argmax(pm, axis=-1)
    v2 = jnp.max(pm, axis=-1)
    topv_ref[...] = jnp.concatenate([v1[:, None], v2[:, None]], axis=1)
    topi_ref[...] = jnp.concatenate([i1[:, None], i2[:, None]], axis=1)
    imp = jnp.sum(p, axis=0, keepdims=True)
    m = jnp.mean(imp)
    var = jnp.sum((imp - m) ** 2) / (E - 1)
    imp_ref[...] = (var / (m * m)).reshape(1, 1)


L = 16


GN_OFF = 0
SB_OFF = L
SR_OFF = L + S_PAD
PACKED = L + 2 * S_PAD
QMAX = (B + GS - 1) // GS


def _route_sc_body(sel_hbm, tv_hbm, outi_hbm, outf_hbm,
                   selv, tvv, permv, outiv, spv, sem1, sem2):
    cid = lax.axis_index("c")
    sid = lax.axis_index("s")

    @pl.when((cid == 0) & (sid == 0))
    def _():
        cp1 = pltpu.make_async_copy(sel_hbm, selv, sem1)
        cp2 = pltpu.make_async_copy(tv_hbm, tvv, sem2)
        cp1.start()
        cp2.start()
        cp1.wait()
        cp2.wait()
        s0 = selv[pl.ds(0, L)]
        s1 = selv[pl.ds(L, L)]
        iota = lax.iota(jnp.int32, L)
        zero = iota * 0
        cnt, rank0, rank1 = [], zero, zero
        for e in range(E):
            m0 = (s0 == e).astype(jnp.int32)
            m1 = (s1 == e).astype(jnp.int32)
            c0 = jnp.sum(m0)
            cnt.append(c0 + jnp.sum(m1))
            rank0 = rank0 + m0 * (plsc.cumsum(m0) - m0)
            rank1 = rank1 + m1 * (plsc.cumsum(m1) - m1 + c0)
        starts, ngr, gstarts = [], [], []
        acc_s = 0
        acc_g = 0
        for e in range(E):
            starts.append(acc_s)
            gstarts.append(acc_g)
            ngr.append((cnt[e] + GS - 1) // GS)
            acc_s = acc_s + cnt[e]
            acc_g = acc_g + ngr[e]
        used = acc_g
        last_e = jnp.int32(0)
        for e in range(E):
            last_e = jnp.where(cnt[e] > 0, jnp.int32(e), last_e)
        st0, st1 = zero, zero
        for e in range(E):
            st0 = st0 + (s0 == e).astype(jnp.int32) * starts[e]
            st1 = st1 + (s1 == e).astype(jnp.int32) * starts[e]
        plsc.store_scatter(permv, [st0 + rank0], iota)
        plsc.store_scatter(permv, [st1 + rank1], iota + L)
        def expert_of(g):
            ev = g * 0
            for e in range(E):
                m = (g >= gstarts[e]) & (g < gstarts[e] + ngr[e])
                ev = ev + m.astype(jnp.int32) * e
            return ev + (g >= used).astype(jnp.int32) * last_e

        gn = zero
        for e in range(E):
            gn = gn + (iota == e).astype(jnp.int32) * gstarts[e]
            gn = gn + (iota == E + e).astype(jnp.int32) * ngr[e]
        outiv[pl.ds(GN_OFF, L)] = gn
        for c in range(S_PAD // L):
            sids = iota + L * c
            gg = sids // GS
            e_s = expert_of(gg)
            gst, cts, sts = zero, zero, zero
            for e in range(E):
                m = (e_s == e).astype(jnp.int32)
                gst = gst + m * gstarts[e]
                cts = cts + m * cnt[e]
                sts = sts + m * starts[e]
            j = (gg - gst) * GS + sids % GS
            valid = (j < cts).astype(jnp.int32)
            sidx = jnp.clip(sts + j, 0, R - 1)
            rows = plsc.load_gather(permv, [sidx])
            probs = plsc.load_gather(tvv, [rows])
            outiv[pl.ds(SR_OFF + L * c, L)] = rows * valid + (1 - valid) * R
            outiv[pl.ds(SB_OFF + L * c, L)] = (rows // NB) * valid
            spv[pl.ds(L * c, L)] = probs * valid.astype(jnp.float32)
        cp3 = pltpu.make_async_copy(outiv, outi_hbm, sem1)
        cp4 = pltpu.make_async_copy(spv, outf_hbm, sem2)
        cp3.start()
        cp4.start()
        cp3.wait()
        cp4.wait()


def _route_sc(sel, beam_scores):
    return pl.kernel(
        _route_sc_body,
        out_type=(
            jax.ShapeDtypeStruct((PACKED,), jnp.int32),
            jax.ShapeDtypeStruct((S_PAD,), jnp.float32),
        ),
        mesh=plsc.VectorSubcoreMesh(
            core_axis_name="c", subcore_axis_name="s", num_cores=1),
        compiler_params=pltpu.CompilerParams(needs_layout_passes=False),
        scratch_types=[
            pltpu.VMEM((R,), jnp.int32),
            pltpu.VMEM((R,), jnp.float32),
            pltpu.VMEM((R,), jnp.int32),
            pltpu.VMEM((PACKED,), jnp.int32),
            pltpu.VMEM((S_PAD,), jnp.float32),
            pltpu.SemaphoreType.DMA,
            pltpu.SemaphoreType.DMA,
        ],
    )(sel, beam_scores)


def _ffn_kernel(ci_ref, sp_ref,
                x_ref, w1_ref, b1_ref, w2_ref, b2_ref, out_ref, xs_ref):
    k = pl.program_id(0)
    e = pl.program_id(1)
    gst = ci_ref[GN_OFF + e]
    ng = ci_ref[GN_OFF + E + e]
    w1b = w1_ref[0].astype(jnp.bfloat16)
    w2b = w2_ref[0].astype(jnp.bfloat16)
    for q in range(QMAX):

        @pl.when(q < ng)
        def _():
            g = gst + q

            @pl.when(k == 0)
            def _():
                xs_ref[g] = jnp.concatenate(
                    [x_ref[ci_ref[SB_OFF + GS * g + s]] for s in range(GS)],
                    axis=0).astype(jnp.bfloat16)

            h = jax.lax.dot_general(
                xs_ref[g], w1b, (((1,), (1,)), ((), ())),
                preferred_element_type=jnp.float32)
            h = jax.nn.gelu(h + b1_ref[0, 0])
            part = jax.lax.dot_general(
                h.astype(jnp.bfloat16), w2b, (((1,), (1,)), ((), ())),
                preferred_element_type=jnp.float32)
            for s in range(GS):
                r = GS * g + s
                row = ci_ref[SR_OFF + r]

                @pl.when((row < R) & (k == 0))
                def _():
                    out_ref[row] = sp_ref[r] * (
                        part[T * s:T * (s + 1)] + b2_ref[0, 0])

                @pl.when((row < R) & (k != 0))
                def _():
                    out_ref[row] = out_ref[row] + (
                        sp_ref[r] * part[T * s:T * (s + 1)])


@jax.jit
def kernel(x, gate_w, w1, b1, w2, b2):
    prob, topv, topi, imp = pl.pallas_call(
        _gate_kernel,
        out_shape=(
            jax.ShapeDtypeStruct((B, E), jnp.float32),
            jax.ShapeDtypeStruct((B, NB), jnp.float32),
            jax.ShapeDtypeStruct((B, NB), jnp.int32),
            jax.ShapeDtypeStruct((1, 1), jnp.float32),
        ),
    )(x, gate_w)

    sel = topi.reshape(R)
    beam_scores = topv.reshape(R)
    expert_route = sel[:, None]
    beam_idx = jnp.arange(R, dtype=jnp.int32)
    importance_loss = imp[0, 0]

    packed_i, slot_prob = _route_sc(sel, beam_scores)

    b1r = b1.reshape(E, 1, F)
    b2r = b2.reshape(E, 1, H)

    grid_spec = pltpu.PrefetchScalarGridSpec(
        num_scalar_prefetch=2,
        grid=(KC, E),
        in_specs=[
            pl.BlockSpec((B, T, H), lambda k, e, ci, sp: (0, 0, 0)),
            pl.BlockSpec((1, FH, H), lambda k, e, ci, sp: (e, k, 0)),
            pl.BlockSpec((1, 1, FH), lambda k, e, ci, sp: (e, 0, k)),
            pl.BlockSpec((1, H, FH), lambda k, e, ci, sp: (e, 0, k)),
            pl.BlockSpec((1, 1, H), lambda k, e, ci, sp: (e, 0, 0)),
        ],
        out_specs=pl.BlockSpec((R, T, H), lambda k, e, ci, sp: (0, 0, 0)),
        scratch_shapes=[pltpu.VMEM((G, GS * T, H), jnp.bfloat16)],
    )
    out = pl.pallas_call(
        _ffn_kernel,
        grid_spec=grid_spec,
        out_shape=jax.ShapeDtypeStruct((R, T, H), jnp.float32),
        compiler_params=pltpu.CompilerParams(
            vmem_limit_bytes=60000 * 1024),
    )(packed_i, slot_prob, x, w1, b1r, w2, b2r)

    return out, beam_scores, expert_route, beam_idx, importance_loss
